# baseline (speedup 1.0000x reference)
import functools

import jax
import jax.numpy as jnp
from jax import lax
from jax.experimental import pallas as pl
from jax.experimental.pallas import tpu as pltpu
from jax.experimental.pallas import tpu_sc as plsc

NC = 2
NS = 16
NW = NC * NS
NSPLIT = 4
K = 1000


def _tc_deinterleave(edge_index):
    E = edge_index.shape[1]
    BK = 128000
    assert E % BK == 0
    grid = (E // BK,)

    def body(ei_ref, row_ref, col_ref):
        row_ref[...] = ei_ref[0].reshape(BK // 128, 128)
        col_ref[...] = ei_ref[1].reshape(BK // 128, 128)

    row2, col2 = pl.pallas_call(
        body,
        grid=grid,
        in_specs=[pl.BlockSpec((2, BK), lambda i: (0, i))],
        out_specs=[pl.BlockSpec((BK // 128, 128), lambda i: (i, 0)),
                   pl.BlockSpec((BK // 128, 128), lambda i: (i, 0))],
        out_shape=[jax.ShapeDtypeStruct((E // 128, 128), jnp.int32),
                   jax.ShapeDtypeStruct((E // 128, 128), jnp.int32)],
    )(edge_index)
    return row2.reshape(E), col2.reshape(E)


def _sc_aggregate(row, col, edge_attr, x, zeros8, zerosn1, ones1):
    E = row.shape[0]
    N, Fx = x.shape
    Fe = edge_attr.shape[1]
    per_w = E // NW
    n_iter = per_w // K
    rows_s = N // NSPLIT

    mesh = plsc.VectorSubcoreMesh(
        core_axis_name="c", subcore_axis_name="s",
        num_cores=NC, num_subcores=NS)

    @functools.partial(
        pl.kernel,
        out_type=(
            jax.ShapeDtypeStruct((NC, N, Fx), jnp.float32),
            jax.ShapeDtypeStruct((NC, N, Fe), jnp.float32),
            jax.ShapeDtypeStruct((NC, N), jnp.float32),
        ),
        mesh=mesh,
        compiler_params=pltpu.CompilerParams(use_tc_tiling_on_sc=False),
        scratch_types=[
            pltpu.VMEM_SHARED((N, Fx), jnp.float32),
            pltpu.VMEM_SHARED((N, Fe), jnp.float32),
            pltpu.VMEM_SHARED((N,), jnp.float32),
            pltpu.VMEM((K,), jnp.int32),
            pltpu.VMEM((K,), jnp.int32),
            pltpu.VMEM((K, Fe), jnp.float32),
            pltpu.VMEM((K, Fx), jnp.float32),
            pltpu.VMEM((K,), jnp.float32),
            pltpu.SemaphoreType.DMA,
        ],
    )
    def sc_agg(row_hbm, col_hbm, attr_hbm, x_hbm, zeros8_hbm, zerosn_hbm,
               ones_hbm,
               accx_out, acce_out, cnt_out,
               accx_sh, acce_sh, cnt_sh,
               row_v, col_v, attr_v, xg_v, ones_v, sem):
        c = lax.axis_index("c")
        s = lax.axis_index("s")
        wid = c * NS + s

        sl = pl.ds(s * rows_s, rows_s)

        @pl.when(s < NSPLIT)
        def _():
            pltpu.sync_copy(zeros8_hbm.at[sl], accx_sh.at[sl])
            pltpu.sync_copy(zeros8_hbm.at[sl], acce_sh.at[sl])


        @pl.when(s == 0)
        def _():
            pltpu.sync_copy(zerosn_hbm, cnt_sh)

        pltpu.sync_copy(ones_hbm, ones_v)
        plsc.subcore_barrier()

        base0 = wid * per_w

        def body(i, carry):
            b = base0 + i * K
            pltpu.sync_copy(row_hbm.at[pl.ds(b, K)], row_v)
            pltpu.sync_copy(col_hbm.at[pl.ds(b, K)], col_v)
            pltpu.sync_copy(attr_hbm.at[pl.ds(b, K)], attr_v)
            pltpu.async_copy(x_hbm.at[row_v], xg_v, sem).wait()
            pltpu.sync_copy(xg_v, accx_sh.at[col_v], add=True)
            pltpu.sync_copy(attr_v, acce_sh.at[col_v], add=True)
            pltpu.sync_copy(ones_v, cnt_sh.at[col_v], add=True)
            return carry

        lax.fori_loop(0, n_iter, body, 0)
        plsc.subcore_barrier()

        @pl.when(s < NSPLIT)
        def _():
            pltpu.sync_copy(accx_sh.at[sl], accx_out.at[c, sl])
            pltpu.sync_copy(acce_sh.at[sl], acce_out.at[c, sl])

        @pl.when(s == 0)
        def _():
            pltpu.sync_copy(cnt_sh, cnt_out.at[c])


    return sc_agg(row, col, edge_attr, x, zeros8, zerosn1, ones1)


def _tc_mlp(x, accx, acce, cnt, u2, W1, b1, W2, b2):
    N, Fx = x.shape
    Fe = acce.shape[2]
    H = W1.shape[0]
    BN = 5120
    grid = ((N + BN - 1) // BN,)

    def body(x_ref, ax_ref, ae_ref, cnt_ref, u_ref, w1_ref, b1_ref,
             w2_ref, b2_ref, out_ref):
        cn = jnp.maximum(cnt_ref[0] + cnt_ref[1], 1.0)
        inv = (1.0 / cn)[:, None]
        mx = (ax_ref[0] + ax_ref[1]) * inv
        me = (ae_ref[0] + ae_ref[1]) * inv
        w1 = w1_ref[...]
        h = (jnp.dot(x_ref[...], w1[:Fx], preferred_element_type=jnp.float32)
             + jnp.dot(mx, w1[Fx:2 * Fx], preferred_element_type=jnp.float32)
             + jnp.dot(me, w1[2 * Fx:2 * Fx + Fe],
                       preferred_element_type=jnp.float32)
             + u_ref[0, 0] * w1[2 * Fx + Fe:] + b1_ref[...])
        h = jnp.maximum(h, 0.0)
        out_ref[...] = (jnp.dot(h, w2_ref[...],
                                preferred_element_type=jnp.float32)
                        + b2_ref[...])

    return pl.pallas_call(
        body,
        grid=grid,
        in_specs=[
            pl.BlockSpec((BN, Fx), lambda i: (i, 0)),
            pl.BlockSpec((NC, BN, Fx), lambda i: (0, i, 0)),
            pl.BlockSpec((NC, BN, Fe), lambda i: (0, i, 0)),
            pl.BlockSpec((NC, BN), lambda i: (0, i)),
            pl.BlockSpec((1, 1), lambda i: (0, 0)),
            pl.BlockSpec((H, H), lambda i: (0, 0)),
            pl.BlockSpec((1, H), lambda i: (0, 0)),
            pl.BlockSpec((H, Fx), lambda i: (0, 0)),
            pl.BlockSpec((1, Fx), lambda i: (0, 0)),
        ],
        out_specs=pl.BlockSpec((BN, Fx), lambda i: (i, 0)),
        out_shape=jax.ShapeDtypeStruct((N, Fx), jnp.float32),
    )(x, accx, acce, cnt, u2, W1, b1, W2, b2)


def kernel(x, edge_index, edge_attr, u, batch, W1, b1, W2, b2):
    N, Fx = x.shape
    zeros8 = jnp.zeros((N, Fx), jnp.float32)
    zerosn = jnp.zeros((N,), jnp.float32)
    ones = jnp.ones((K,), jnp.float32)
    row, col = _tc_deinterleave(edge_index)
    accx, acce, cnt = _sc_aggregate(row, col, edge_attr, x,
                                    zeros8, zerosn, ones)
    return _tc_mlp(x, accx, acce, cnt, u.reshape(1, 1),
                   W1, b1.reshape(1, -1), W2, b2.reshape(1, -1))


# speedup vs baseline: 1.0010x; 1.0010x over previous
import functools

import jax
import jax.numpy as jnp
from jax import lax
from jax.experimental import pallas as pl
from jax.experimental.pallas import tpu as pltpu
from jax.experimental.pallas import tpu_sc as plsc

NC = 2
NS = 16
NW = NC * NS
NSPLIT = 4
K = 1000


def _tc_deinterleave(edge_index):
    E = edge_index.shape[1]
    BK = 128000
    assert E % BK == 0
    grid = (E // BK,)

    def body(ei_ref, row_ref, col_ref):
        row_ref[...] = ei_ref[0].reshape(BK // 128, 128)
        col_ref[...] = ei_ref[1].reshape(BK // 128, 128)

    row2, col2 = pl.pallas_call(
        body,
        grid=grid,
        in_specs=[pl.BlockSpec((2, BK), lambda i: (0, i))],
        out_specs=[pl.BlockSpec((BK // 128, 128), lambda i: (i, 0)),
                   pl.BlockSpec((BK // 128, 128), lambda i: (i, 0))],
        out_shape=[jax.ShapeDtypeStruct((E // 128, 128), jnp.int32),
                   jax.ShapeDtypeStruct((E // 128, 128), jnp.int32)],
    )(edge_index)
    return lax.optimization_barrier((row2.reshape(E), col2.reshape(E)))


def _sc_aggregate(row, col, edge_attr, x, zeros8, zerosn1, ones1):
    E = row.shape[0]
    N, Fx = x.shape
    Fe = edge_attr.shape[1]
    per_w = E // NW
    n_iter = per_w // K
    rows_s = N // NSPLIT

    mesh = plsc.VectorSubcoreMesh(
        core_axis_name="c", subcore_axis_name="s",
        num_cores=NC, num_subcores=NS)

    @functools.partial(
        pl.kernel,
        out_type=(
            jax.ShapeDtypeStruct((NC, N, Fx), jnp.float32),
            jax.ShapeDtypeStruct((NC, N, Fe), jnp.float32),
            jax.ShapeDtypeStruct((NC, N), jnp.float32),
        ),
        mesh=mesh,
        compiler_params=pltpu.CompilerParams(use_tc_tiling_on_sc=False),
        scratch_types=[
            pltpu.VMEM_SHARED((N, Fx), jnp.float32),
            pltpu.VMEM_SHARED((N, Fe), jnp.float32),
            pltpu.VMEM_SHARED((N,), jnp.float32),
            pltpu.VMEM((K,), jnp.int32),
            pltpu.VMEM((K,), jnp.int32),
            pltpu.VMEM((K, Fe), jnp.float32),
            pltpu.VMEM((K, Fx), jnp.float32),
            pltpu.VMEM((K,), jnp.float32),
            pltpu.SemaphoreType.DMA,
        ],
    )
    def sc_agg(row_hbm, col_hbm, attr_hbm, x_hbm, zeros8_hbm, zerosn_hbm,
               ones_hbm,
               accx_out, acce_out, cnt_out,
               accx_sh, acce_sh, cnt_sh,
               row_v, col_v, attr_v, xg_v, ones_v, sem):
        c = lax.axis_index("c")
        s = lax.axis_index("s")
        wid = c * NS + s

        sl = pl.ds(s * rows_s, rows_s)

        @pl.when(s < NSPLIT)
        def _():
            pltpu.sync_copy(zeros8_hbm.at[sl], accx_sh.at[sl])
            pltpu.sync_copy(zeros8_hbm.at[sl], acce_sh.at[sl])


        @pl.when(s == 0)
        def _():
            pltpu.sync_copy(zerosn_hbm, cnt_sh)

        pltpu.sync_copy(ones_hbm, ones_v)
        plsc.subcore_barrier()

        base0 = wid * per_w

        def body(i, carry):
            b = base0 + i * K
            pltpu.sync_copy(row_hbm.at[pl.ds(b, K)], row_v)
            pltpu.sync_copy(col_hbm.at[pl.ds(b, K)], col_v)
            pltpu.sync_copy(attr_hbm.at[pl.ds(b, K)], attr_v)
            pltpu.async_copy(x_hbm.at[row_v], xg_v, sem).wait()
            pltpu.sync_copy(xg_v, accx_sh.at[col_v], add=True)
            pltpu.sync_copy(attr_v, acce_sh.at[col_v], add=True)
            pltpu.sync_copy(ones_v, cnt_sh.at[col_v], add=True)
            return carry

        lax.fori_loop(0, n_iter, body, 0)
        plsc.subcore_barrier()

        @pl.when(s < NSPLIT)
        def _():
            pltpu.sync_copy(accx_sh.at[sl], accx_out.at[c, sl])
            pltpu.sync_copy(acce_sh.at[sl], acce_out.at[c, sl])

        @pl.when(s == 0)
        def _():
            pltpu.sync_copy(cnt_sh, cnt_out.at[c])


    return sc_agg(row, col, edge_attr, x, zeros8, zerosn1, ones1)


def _tc_mlp(x, accx, acce, cnt, u2, W1, b1, W2, b2):
    N, Fx = x.shape
    Fe = acce.shape[2]
    H = W1.shape[0]
    BN = 5120
    grid = ((N + BN - 1) // BN,)

    def body(x_ref, ax_ref, ae_ref, cnt_ref, u_ref, w1_ref, b1_ref,
             w2_ref, b2_ref, out_ref):
        cn = jnp.maximum(cnt_ref[0] + cnt_ref[1], 1.0)
        inv = (1.0 / cn)[:, None]
        mx = (ax_ref[0] + ax_ref[1]) * inv
        me = (ae_ref[0] + ae_ref[1]) * inv
        w1 = w1_ref[...]
        h = (jnp.dot(x_ref[...], w1[:Fx], preferred_element_type=jnp.float32)
             + jnp.dot(mx, w1[Fx:2 * Fx], preferred_element_type=jnp.float32)
             + jnp.dot(me, w1[2 * Fx:2 * Fx + Fe],
                       preferred_element_type=jnp.float32)
             + u_ref[0, 0] * w1[2 * Fx + Fe:] + b1_ref[...])
        h = jnp.maximum(h, 0.0)
        out_ref[...] = (jnp.dot(h, w2_ref[...],
                                preferred_element_type=jnp.float32)
                        + b2_ref[...])

    return pl.pallas_call(
        body,
        grid=grid,
        in_specs=[
            pl.BlockSpec((BN, Fx), lambda i: (i, 0)),
            pl.BlockSpec((NC, BN, Fx), lambda i: (0, i, 0)),
            pl.BlockSpec((NC, BN, Fe), lambda i: (0, i, 0)),
            pl.BlockSpec((NC, BN), lambda i: (0, i)),
            pl.BlockSpec((1, 1), lambda i: (0, 0)),
            pl.BlockSpec((H, H), lambda i: (0, 0)),
            pl.BlockSpec((1, H), lambda i: (0, 0)),
            pl.BlockSpec((H, Fx), lambda i: (0, 0)),
            pl.BlockSpec((1, Fx), lambda i: (0, 0)),
        ],
        out_specs=pl.BlockSpec((BN, Fx), lambda i: (i, 0)),
        out_shape=jax.ShapeDtypeStruct((N, Fx), jnp.float32),
    )(x, accx, acce, cnt, u2, W1, b1, W2, b2)


def kernel(x, edge_index, edge_attr, u, batch, W1, b1, W2, b2):
    N, Fx = x.shape
    zeros8 = jnp.zeros((N, Fx), jnp.float32)
    zerosn = jnp.zeros((N,), jnp.float32)
    ones = jnp.ones((K,), jnp.float32)
    row, col = _tc_deinterleave(edge_index)
    accx, acce, cnt = _sc_aggregate(row, col, edge_attr, x,
                                    zeros8, zerosn, ones)
    return _tc_mlp(x, accx, acce, cnt, u.reshape(1, 1),
                   W1, b1.reshape(1, -1), W2, b2.reshape(1, -1))


# speedup vs baseline: 1.8532x; 1.8513x over previous
import functools

import jax
import jax.numpy as jnp
from jax import lax
from jax.experimental import pallas as pl
from jax.experimental.pallas import tpu as pltpu
from jax.experimental.pallas import tpu_sc as plsc

NC = 2
NS = 16
NW = NC * NS
NSPLIT = 4
K = 640
KT = K // 128


def _tc_deinterleave(edge_index):
    E = edge_index.shape[1]
    BK = 128000
    assert E % BK == 0
    grid = (E // BK,)

    def body(ei_ref, row_ref, col_ref):
        row_ref[...] = ei_ref[0].reshape(BK // 128, 128)
        col_ref[...] = ei_ref[1].reshape(BK // 128, 128)

    row2, col2 = pl.pallas_call(
        body,
        grid=grid,
        in_specs=[pl.BlockSpec((2, BK), lambda i: (0, i))],
        out_specs=[pl.BlockSpec((BK // 128, 128), lambda i: (i, 0)),
                   pl.BlockSpec((BK // 128, 128), lambda i: (i, 0))],
        out_shape=[jax.ShapeDtypeStruct((E // 128, 128), jnp.int32),
                   jax.ShapeDtypeStruct((E // 128, 128), jnp.int32)],
    )(edge_index)
    return lax.optimization_barrier((row2.reshape(E), col2.reshape(E)))


def _sc_aggregate(row, col, attr3, x, zeros8, zerosn1, ones1):
    E = row.shape[0]
    N, Fx = x.shape
    Fe = attr3.shape[1]
    n_chunks = E // K
    n_full, n_rem = divmod(n_chunks, NW)
    rows_s = N // NSPLIT

    mesh = plsc.VectorSubcoreMesh(
        core_axis_name="c", subcore_axis_name="s",
        num_cores=NC, num_subcores=NS)

    @functools.partial(
        pl.kernel,
        out_type=(
            jax.ShapeDtypeStruct((NC, N, Fx), jnp.float32),
            jax.ShapeDtypeStruct((NC, N, Fe), jnp.float32),
            jax.ShapeDtypeStruct((NC, N), jnp.float32),
        ),
        mesh=mesh,
        compiler_params=pltpu.CompilerParams(use_tc_tiling_on_sc=False,
                                             needs_layout_passes=False),
        scratch_types=[
            pltpu.VMEM_SHARED((N, Fx), jnp.float32),
            pltpu.VMEM_SHARED((N, Fe), jnp.float32),
            pltpu.VMEM_SHARED((N,), jnp.float32),
            pltpu.VMEM((K,), jnp.int32),
            pltpu.VMEM((K,), jnp.int32),
            pltpu.VMEM((KT, Fe, 128), jnp.float32),
            pltpu.VMEM((K, Fe), jnp.float32),
            pltpu.VMEM((K, Fx), jnp.float32),
            pltpu.VMEM((K,), jnp.float32),
            pltpu.SemaphoreType.DMA,
        ],
    )
    def sc_agg(row_hbm, col_hbm, attr_hbm, x_hbm, zeros8_hbm, zerosn_hbm,
               ones_hbm,
               accx_out, acce_out, cnt_out,
               accx_sh, acce_sh, cnt_sh,
               row_v, col_v, attr_v, attr_r, xg_v, ones_v, sem):
        c = lax.axis_index("c")
        s = lax.axis_index("s")
        wid = c * NS + s

        sl = pl.ds(s * rows_s, rows_s)

        @pl.when(s < NSPLIT)
        def _():
            pltpu.sync_copy(zeros8_hbm.at[sl], accx_sh.at[sl])
            pltpu.sync_copy(zeros8_hbm.at[sl], acce_sh.at[sl])


        @pl.when(s == 0)
        def _():
            pltpu.sync_copy(zerosn_hbm, cnt_sh)

        pltpu.sync_copy(ones_hbm, ones_v)
        plsc.subcore_barrier()

        def do_chunk(ch):
            iota16 = lax.iota(jnp.int32, 16)
            b = ch * K
            pltpu.sync_copy(row_hbm.at[pl.ds(b, K)], row_v)
            pltpu.sync_copy(col_hbm.at[pl.ds(b, K)], col_v)
            pltpu.sync_copy(attr_hbm.at[pl.ds(ch * KT, KT)], attr_v)
            pltpu.async_copy(x_hbm.at[row_v], xg_v, sem).wait()
            for t in range(KT):
                for f in range(Fe):
                    for jj in range(0, 128, 16):
                        val = attr_v[t, f, pl.ds(jj, 16)]
                        ridx = iota16 + (t * 128 + jj)
                        plsc.store_scatter(attr_r, [ridx, iota16 * 0 + f],
                                           val)
            pltpu.sync_copy(xg_v, accx_sh.at[col_v], add=True)
            pltpu.sync_copy(attr_r, acce_sh.at[col_v], add=True)
            pltpu.sync_copy(ones_v, cnt_sh.at[col_v], add=True)

        def body(i, carry):
            do_chunk(wid + i * NW)
            return carry

        lax.fori_loop(0, n_full, body, 0)

        @pl.when(wid < n_rem)
        def _():
            do_chunk(wid + n_full * NW)
        plsc.subcore_barrier()

        @pl.when(s < NSPLIT)
        def _():
            pltpu.sync_copy(accx_sh.at[sl], accx_out.at[c, sl])
            pltpu.sync_copy(acce_sh.at[sl], acce_out.at[c, sl])

        @pl.when(s == 0)
        def _():
            pltpu.sync_copy(cnt_sh, cnt_out.at[c])


    return sc_agg(row, col, attr3, x, zeros8, zerosn1, ones1)


def _tc_mlp(x, accx, acce, cnt, u2, W1, b1, W2, b2):
    N, Fx = x.shape
    Fe = acce.shape[2]
    H = W1.shape[0]
    BN = 5120
    grid = ((N + BN - 1) // BN,)

    def body(x_ref, ax_ref, ae_ref, cnt_ref, u_ref, w1_ref, b1_ref,
             w2_ref, b2_ref, out_ref):
        cn = jnp.maximum(cnt_ref[0] + cnt_ref[1], 1.0)
        inv = (1.0 / cn)[:, None]
        mx = (ax_ref[0] + ax_ref[1]) * inv
        me = (ae_ref[0] + ae_ref[1]) * inv
        w1 = w1_ref[...]
        h = (jnp.dot(x_ref[...], w1[:Fx], preferred_element_type=jnp.float32)
             + jnp.dot(mx, w1[Fx:2 * Fx], preferred_element_type=jnp.float32)
             + jnp.dot(me, w1[2 * Fx:2 * Fx + Fe],
                       preferred_element_type=jnp.float32)
             + u_ref[0, 0] * w1[2 * Fx + Fe:] + b1_ref[...])
        h = jnp.maximum(h, 0.0)
        out_ref[...] = (jnp.dot(h, w2_ref[...],
                                preferred_element_type=jnp.float32)
                        + b2_ref[...])

    return pl.pallas_call(
        body,
        grid=grid,
        in_specs=[
            pl.BlockSpec((BN, Fx), lambda i: (i, 0)),
            pl.BlockSpec((NC, BN, Fx), lambda i: (0, i, 0)),
            pl.BlockSpec((NC, BN, Fe), lambda i: (0, i, 0)),
            pl.BlockSpec((NC, BN), lambda i: (0, i)),
            pl.BlockSpec((1, 1), lambda i: (0, 0)),
            pl.BlockSpec((H, H), lambda i: (0, 0)),
            pl.BlockSpec((1, H), lambda i: (0, 0)),
            pl.BlockSpec((H, Fx), lambda i: (0, 0)),
            pl.BlockSpec((1, Fx), lambda i: (0, 0)),
        ],
        out_specs=pl.BlockSpec((BN, Fx), lambda i: (i, 0)),
        out_shape=jax.ShapeDtypeStruct((N, Fx), jnp.float32),
    )(x, accx, acce, cnt, u2, W1, b1, W2, b2)


def kernel(x, edge_index, edge_attr, u, batch, W1, b1, W2, b2):
    N, Fx = x.shape
    zeros8 = jnp.zeros((N, Fx), jnp.float32)
    zerosn = jnp.zeros((N,), jnp.float32)
    ones = jnp.ones((K,), jnp.float32)
    row, col = _tc_deinterleave(edge_index)
    E = edge_index.shape[1]
    attr3 = edge_attr.reshape(E // 128, 128, -1).transpose(0, 2, 1)
    attr3 = lax.optimization_barrier(attr3)
    accx, acce, cnt = _sc_aggregate(row, col, attr3, x,
                                    zeros8, zerosn, ones)
    return _tc_mlp(x, accx, acce, cnt, u.reshape(1, 1),
                   W1, b1.reshape(1, -1), W2, b2.reshape(1, -1))


# speedup vs baseline: 3.0080x; 1.6231x over previous
import functools

import jax
import jax.numpy as jnp
from jax import lax
from jax.experimental import pallas as pl
from jax.experimental.pallas import tpu as pltpu
from jax.experimental.pallas import tpu_sc as plsc

NC = 2
NS = 16
NW = NC * NS
NSPLIT = 4
K = 640
KT = K // 128


def _tc_deinterleave(edge_index):
    E = edge_index.shape[1]
    BK = 128000
    assert E % BK == 0
    grid = (E // BK,)

    def body(ei_ref, row_ref, col_ref):
        row_ref[...] = ei_ref[0].reshape(BK // 128, 128)
        col_ref[...] = ei_ref[1].reshape(BK // 128, 128)

    row2, col2 = pl.pallas_call(
        body,
        grid=grid,
        in_specs=[pl.BlockSpec((2, BK), lambda i: (0, i))],
        out_specs=[pl.BlockSpec((BK // 128, 128), lambda i: (i, 0)),
                   pl.BlockSpec((BK // 128, 128), lambda i: (i, 0))],
        out_shape=[jax.ShapeDtypeStruct((E // 128, 128), jnp.int32),
                   jax.ShapeDtypeStruct((E // 128, 128), jnp.int32)],
    )(edge_index)
    return lax.optimization_barrier((row2.reshape(E), col2.reshape(E)))


def _sc_aggregate(row, col, attr3, x, zeros8, zerosn1, ones1):
    E = row.shape[0]
    N, Fx = x.shape
    Fe = attr3.shape[1]
    n_chunks = E // K
    n_full, n_rem = divmod(n_chunks, NW)
    rows_s = N // NSPLIT

    mesh = plsc.VectorSubcoreMesh(
        core_axis_name="c", subcore_axis_name="s",
        num_cores=NC, num_subcores=NS)

    @functools.partial(
        pl.kernel,
        out_type=(
            jax.ShapeDtypeStruct((NC, N, Fx), jnp.float32),
            jax.ShapeDtypeStruct((NC, N, Fe), jnp.float32),
            jax.ShapeDtypeStruct((NC, N), jnp.float32),
        ),
        mesh=mesh,
        compiler_params=pltpu.CompilerParams(use_tc_tiling_on_sc=False,
                                             needs_layout_passes=False),
        scratch_types=[
            pltpu.VMEM_SHARED((N, Fx), jnp.float32),
            pltpu.VMEM_SHARED((N, Fe), jnp.float32),
            pltpu.VMEM_SHARED((N,), jnp.float32),
            pltpu.VMEM((K,), jnp.int32),
            pltpu.VMEM((K,), jnp.int32),
            pltpu.VMEM((KT, Fe, 128), jnp.float32),
            pltpu.VMEM((K, Fe), jnp.float32),
            pltpu.VMEM((K, Fx), jnp.float32),
            pltpu.VMEM((K,), jnp.float32),
            pltpu.SemaphoreType.DMA,
            pltpu.SemaphoreType.DMA,
            pltpu.SemaphoreType.DMA,
            pltpu.SemaphoreType.DMA,
            pltpu.SemaphoreType.DMA,
            pltpu.SemaphoreType.DMA,
            pltpu.SemaphoreType.DMA,
        ],
    )
    def sc_agg(row_hbm, col_hbm, attr_hbm, x_hbm, zeros8_hbm, zerosn_hbm,
               ones_hbm,
               accx_out, acce_out, cnt_out,
               accx_sh, acce_sh, cnt_sh,
               row_v, col_v, attr_v, attr_r, xg_v, ones_v,
               sem_r, sem_c, sem_a, sem_g, sem_s1, sem_s2, sem_s3):
        c = lax.axis_index("c")
        s = lax.axis_index("s")
        wid = c * NS + s

        sl = pl.ds(s * rows_s, rows_s)

        @pl.when(s < NSPLIT)
        def _():
            pltpu.sync_copy(zeros8_hbm.at[sl], accx_sh.at[sl])
            pltpu.sync_copy(zeros8_hbm.at[sl], acce_sh.at[sl])


        @pl.when(s == 0)
        def _():
            pltpu.sync_copy(zerosn_hbm, cnt_sh)

        pltpu.sync_copy(ones_hbm, ones_v)
        plsc.subcore_barrier()

        def do_chunk(ch):
            iota16 = lax.iota(jnp.int32, 16)
            b = ch * K
            d_r = pltpu.async_copy(row_hbm.at[pl.ds(b, K)], row_v, sem_r)
            d_c = pltpu.async_copy(col_hbm.at[pl.ds(b, K)], col_v, sem_c)
            d_a = pltpu.async_copy(attr_hbm.at[pl.ds(ch * KT, KT)], attr_v,
                                   sem_a)
            d_r.wait()
            d_g = pltpu.async_copy(x_hbm.at[row_v], xg_v, sem_g)
            d_a.wait()
            for t in range(KT):
                for f in range(Fe):
                    for jj in range(0, 128, 16):
                        val = attr_v[t, f, pl.ds(jj, 16)]
                        ridx = iota16 + (t * 128 + jj)
                        plsc.store_scatter(attr_r, [ridx, iota16 * 0 + f],
                                           val)
            d_c.wait()
            d_g.wait()
            d1 = pltpu.async_copy(xg_v, accx_sh.at[col_v], sem_s1, add=True)
            d2 = pltpu.async_copy(attr_r, acce_sh.at[col_v], sem_s2, add=True)
            d3 = pltpu.async_copy(ones_v, cnt_sh.at[col_v], sem_s3, add=True)
            d1.wait()
            d2.wait()
            d3.wait()

        def body(i, carry):
            do_chunk(wid + i * NW)
            return carry

        lax.fori_loop(0, n_full, body, 0)

        @pl.when(wid < n_rem)
        def _():
            do_chunk(wid + n_full * NW)
        plsc.subcore_barrier()

        @pl.when(s < NSPLIT)
        def _():
            pltpu.sync_copy(accx_sh.at[sl], accx_out.at[c, sl])
            pltpu.sync_copy(acce_sh.at[sl], acce_out.at[c, sl])

        @pl.when(s == 0)
        def _():
            pltpu.sync_copy(cnt_sh, cnt_out.at[c])


    return sc_agg(row, col, attr3, x, zeros8, zerosn1, ones1)


def _tc_mlp(x, accx, acce, cnt, u2, W1, b1, W2, b2):
    N, Fx = x.shape
    Fe = acce.shape[2]
    H = W1.shape[0]
    BN = 5120
    grid = ((N + BN - 1) // BN,)

    def body(x_ref, ax_ref, ae_ref, cnt_ref, u_ref, w1_ref, b1_ref,
             w2_ref, b2_ref, out_ref):
        cn = jnp.maximum(cnt_ref[0] + cnt_ref[1], 1.0)
        inv = (1.0 / cn)[:, None]
        mx = (ax_ref[0] + ax_ref[1]) * inv
        me = (ae_ref[0] + ae_ref[1]) * inv
        w1 = w1_ref[...]
        h = (jnp.dot(x_ref[...], w1[:Fx], preferred_element_type=jnp.float32)
             + jnp.dot(mx, w1[Fx:2 * Fx], preferred_element_type=jnp.float32)
             + jnp.dot(me, w1[2 * Fx:2 * Fx + Fe],
                       preferred_element_type=jnp.float32)
             + u_ref[0, 0] * w1[2 * Fx + Fe:] + b1_ref[...])
        h = jnp.maximum(h, 0.0)
        out_ref[...] = (jnp.dot(h, w2_ref[...],
                                preferred_element_type=jnp.float32)
                        + b2_ref[...])

    return pl.pallas_call(
        body,
        grid=grid,
        in_specs=[
            pl.BlockSpec((BN, Fx), lambda i: (i, 0)),
            pl.BlockSpec((NC, BN, Fx), lambda i: (0, i, 0)),
            pl.BlockSpec((NC, BN, Fe), lambda i: (0, i, 0)),
            pl.BlockSpec((NC, BN), lambda i: (0, i)),
            pl.BlockSpec((1, 1), lambda i: (0, 0)),
            pl.BlockSpec((H, H), lambda i: (0, 0)),
            pl.BlockSpec((1, H), lambda i: (0, 0)),
            pl.BlockSpec((H, Fx), lambda i: (0, 0)),
            pl.BlockSpec((1, Fx), lambda i: (0, 0)),
        ],
        out_specs=pl.BlockSpec((BN, Fx), lambda i: (i, 0)),
        out_shape=jax.ShapeDtypeStruct((N, Fx), jnp.float32),
    )(x, accx, acce, cnt, u2, W1, b1, W2, b2)


def kernel(x, edge_index, edge_attr, u, batch, W1, b1, W2, b2):
    N, Fx = x.shape
    zeros8 = jnp.zeros((N, Fx), jnp.float32)
    zerosn = jnp.zeros((N,), jnp.float32)
    ones = jnp.ones((K,), jnp.float32)
    row, col = _tc_deinterleave(edge_index)
    E = edge_index.shape[1]
    attr3 = edge_attr.reshape(E // 128, 128, -1).transpose(0, 2, 1)
    attr3 = lax.optimization_barrier(attr3)
    accx, acce, cnt = _sc_aggregate(row, col, attr3, x,
                                    zeros8, zerosn, ones)
    return _tc_mlp(x, accx, acce, cnt, u.reshape(1, 1),
                   W1, b1.reshape(1, -1), W2, b2.reshape(1, -1))
